# Initial kernel scaffold; baseline (speedup 1.0000x reference)
import jax
import jax.numpy as jnp
from jax.experimental import pallas as pl


def kernel(x, embL, embQ):
    raise NotImplementedError("write your pallas kernel here")


# speedup vs baseline: 2.1700x; 2.1700x over previous
import jax
import jax.numpy as jnp
from jax import lax
from jax.experimental import pallas as pl
from jax.experimental.pallas import tpu as pltpu
from jax.experimental.pallas import tpu_sc as plsc

B = 16384
F = 26
D = 32
V = 1000000
NC, NS = 2, 16
NW = NC * NS
BPW = B // NW
C = 64
NCHUNK = BPW // C
IPC = C * F
GW = 128
NG = IPC // GW

_GATHER_DNUMS = lax.GatherDimensionNumbers(
    offset_dims=(), collapsed_slice_dims=(0,), start_index_map=(0,))


def _lane_gather(t, perm):
    return lax.gather(t, perm[:, None], _GATHER_DNUMS, (1,),
                      mode=lax.GatherScatterMode.PROMISE_IN_BOUNDS)


def _fm_body(x_hbm, embL_hbm, embQ_hbm, out_hbm,
             idx_v, rowsQ, eL_v, out_v, sems, semL):
    c = lax.axis_index("c")
    s = lax.axis_index("s")
    wid = s * NC + c
    iota = lax.iota(jnp.int32, 16)
    tail_mask = iota < (F - 16)
    zero = jnp.zeros((16,), jnp.float32)

    def stage_and_fire(ci, buf):
        base = wid * BPW + ci * C
        pltpu.sync_copy(x_hbm.at[pl.ds(base * F, IPC)], idx_v.at[buf])
        for j in range(NG):
            idx_j = idx_v.at[buf].at[pl.ds(j * GW, GW)]
            pltpu.async_copy(
                embQ_hbm.at[idx_j],
                rowsQ.at[buf].at[pl.ds(j * GW, GW)], sems.at[buf])
            pltpu.async_copy(
                embL_hbm.at[idx_j],
                eL_v.at[buf].at[pl.ds(j * GW, GW)], semL.at[buf])

    def drain(buf):
        pltpu.make_async_copy(
            embQ_hbm.at[pl.ds(0, IPC)], rowsQ.at[buf], sems.at[buf]).wait()
        pltpu.make_async_copy(
            embL_hbm.at[pl.ds(0, IPC)],
            eL_v.at[buf].at[pl.ds(0, IPC)], semL.at[buf]).wait()

    def compute(ci, buf):
        base = wid * BPW + ci * C
        for g in range(C // 16):
            def row_body(j, ov):
                i0 = (g * 16 + j) * F
                z0 = z1 = s0 = s1 = zero
                for f in range(F):
                    v0 = rowsQ[buf, i0 + f, pl.ds(0, 16)]
                    v1 = rowsQ[buf, i0 + f, pl.ds(16, 16)]
                    z0 = z0 + v0
                    z1 = z1 + v1
                    s0 = s0 + v0 * v0
                    s1 = s1 + v1 * v1
                l0 = eL_v[buf, pl.ds(i0, 16)]
                l1 = jnp.where(tail_mask, eL_v[buf, pl.ds(i0 + 16, 16)], 0.0)
                t = 0.5 * ((s0 - z0 * z0) + (s1 - z1 * z1)) + l0 + l1
                for k in (8, 4, 2, 1):
                    t = t + _lane_gather(t, iota ^ k)
                return jnp.where(iota == j, t, ov)

            ov = lax.fori_loop(0, 16, row_body, zero)
            out_v[pl.ds(g * 16, 16)] = ov
        pltpu.sync_copy(out_v, out_hbm.at[pl.ds(base, C)])

    stage_and_fire(0, 0)

    def pair_body(p, carry):
        ci0 = p * 2
        stage_and_fire(ci0 + 1, 1)
        drain(0)
        compute(ci0, 0)

        @pl.when(ci0 + 2 < NCHUNK)
        def _():
            stage_and_fire(ci0 + 2, 0)
        drain(1)
        compute(ci0 + 1, 1)
        return carry

    lax.fori_loop(0, NCHUNK // 2, pair_body, 0)


@jax.jit
def kernel(x, embL, embQ):
    x_flat = x.reshape(B * F).astype(jnp.int32)
    embL_flat = embL.reshape(-1)
    mesh = plsc.VectorSubcoreMesh(
        core_axis_name="c", subcore_axis_name="s",
        num_cores=NC, num_subcores=NS)
    fm = pl.kernel(
        _fm_body,
        out_type=jax.ShapeDtypeStruct((B,), jnp.float32),
        mesh=mesh,
        scratch_types=[
            pltpu.VMEM((2, IPC), jnp.int32),
            pltpu.VMEM((2, IPC, D), jnp.float32),
            pltpu.VMEM((2, IPC + 16), jnp.float32),
            pltpu.VMEM((C,), jnp.float32),
            pltpu.SemaphoreType.DMA((2,)),
            pltpu.SemaphoreType.DMA((2,)),
        ],
        compiler_params=pltpu.CompilerParams(use_tc_tiling_on_sc=False),
    )
    return fm(x_flat, embL_flat, embQ)
